# Initial kernel scaffold; baseline (speedup 1.0000x reference)
import jax
import jax.numpy as jnp
from jax.experimental import pallas as pl


def kernel(input, tables):
    raise NotImplementedError("write your pallas kernel here")


# speedup vs baseline: 3.3852x; 3.3852x over previous
import functools

import jax
import jax.numpy as jnp
from jax import lax
from jax.experimental import pallas as pl
from jax.experimental.pallas import tpu as pltpu
from jax.experimental.pallas import tpu_sc as plsc

_NC = 2
_NS = 16
_NW = _NC * _NS

_LANES = 16


def _gather_kernel(n_total, d, v_rows, f_fields, chunk):
    n_per_w = n_total // _NW
    n_chunks = n_per_w // chunk
    period = 208
    assert chunk % period == 0 and n_per_w % chunk == 0

    mesh = plsc.VectorSubcoreMesh(core_axis_name="c", subcore_axis_name="s")

    @functools.partial(
        pl.kernel,
        mesh=mesh,
        compiler_params=pltpu.CompilerParams(use_tc_tiling_on_sc=False),
        out_type=jax.ShapeDtypeStruct((n_total, d), jnp.float32),
        scratch_types=[
            pltpu.VMEM((chunk,), jnp.int32),
            pltpu.VMEM((chunk, d), jnp.float32),
            pltpu.SemaphoreType.DMA,
        ],
    )
    def k(tbl_hbm, idx_hbm, out_hbm, idx_v, rows_v, sem):
        cid = lax.axis_index("c")
        sid = lax.axis_index("s")
        wid = sid * _NC + cid
        base0 = wid * n_per_w

        offs = []
        for j in range(period // _LANES):
            lane = lax.iota(jnp.int32, _LANES)
            offs.append(((lane + j * _LANES) % f_fields) * v_rows)

        def body(i, carry):
            base = base0 + i * chunk
            pltpu.sync_copy(idx_hbm.at[pl.ds(base, chunk)], idx_v)
            for j in range(chunk // _LANES):
                sl = pl.ds(j * _LANES, _LANES)
                idx_v[sl] = idx_v[sl] + offs[j % (period // _LANES)]
            pltpu.async_copy(tbl_hbm.at[idx_v], rows_v, sem).wait()
            pltpu.sync_copy(rows_v, out_hbm.at[pl.ds(base, chunk)])
            return carry

        lax.fori_loop(0, n_chunks, body, 0, unroll=False)

    return k


def kernel(input, tables):
    b, l, f = input.shape
    f2, v, d = tables.shape
    n = b * l * f
    idx_flat = input.reshape(n)
    tbl = tables.reshape(f2 * v, d)
    out = _gather_kernel(n, d, v, f, 1664)(tbl, idx_flat)
    return out.reshape(b, l, f, d)


# speedup vs baseline: 3.4153x; 1.0089x over previous
import functools

import jax
import jax.numpy as jnp
from jax import lax
from jax.experimental import pallas as pl
from jax.experimental.pallas import tpu as pltpu
from jax.experimental.pallas import tpu_sc as plsc

_NC = 2
_NS = 16
_NW = _NC * _NS

_LANES = 16


def _gather_kernel(n_total, d, v_rows, f_fields, chunk):
    n_per_w = n_total // _NW
    n_chunks = n_per_w // chunk
    period = 208
    assert chunk % period == 0 and n_per_w % chunk == 0
    assert n_chunks % 2 == 0 and n_chunks >= 4

    mesh = plsc.VectorSubcoreMesh(core_axis_name="c", subcore_axis_name="s")

    @functools.partial(
        pl.kernel,
        mesh=mesh,
        compiler_params=pltpu.CompilerParams(use_tc_tiling_on_sc=False),
        out_type=jax.ShapeDtypeStruct((n_total, d), jnp.float32),
        scratch_types=[
            pltpu.VMEM((chunk,), jnp.int32),
            pltpu.VMEM((chunk,), jnp.int32),
            pltpu.VMEM((chunk, d), jnp.float32),
            pltpu.VMEM((chunk, d), jnp.float32),
            pltpu.SemaphoreType.DMA,
            pltpu.SemaphoreType.DMA,
        ],
    )
    def k(tbl_hbm, idx_hbm, out_hbm, idx0, idx1, rows0, rows1, sem0, sem1):
        cid = lax.axis_index("c")
        sid = lax.axis_index("s")
        wid = sid * _NC + cid
        base0 = wid * n_per_w

        offs = []
        for j in range(period // _LANES):
            lane = lax.iota(jnp.int32, _LANES)
            offs.append(((lane + j * _LANES) % f_fields) * v_rows)

        def load_idx(ci, idx_v):
            pltpu.sync_copy(idx_hbm.at[pl.ds(base0 + ci * chunk, chunk)], idx_v)
            for j in range(chunk // _LANES):
                sl = pl.ds(j * _LANES, _LANES)
                idx_v[sl] = idx_v[sl] + offs[j % (period // _LANES)]

        def fire_gather(idx_v, rows_v, sem):
            pltpu.make_async_copy(tbl_hbm.at[idx_v], rows_v, sem).start()

        def wait_gather(idx_v, rows_v, sem):
            pltpu.make_async_copy(tbl_hbm.at[idx_v], rows_v, sem).wait()

        def writeout(ci, rows_v):
            pltpu.sync_copy(rows_v, out_hbm.at[pl.ds(base0 + ci * chunk, chunk)])

        load_idx(0, idx0)
        fire_gather(idx0, rows0, sem0)

        def body(g, carry):
            j = 2 * g
            load_idx(j + 1, idx1)
            fire_gather(idx1, rows1, sem1)
            wait_gather(idx0, rows0, sem0)
            writeout(j, rows0)
            load_idx(j + 2, idx0)
            fire_gather(idx0, rows0, sem0)
            wait_gather(idx1, rows1, sem1)
            writeout(j + 1, rows1)
            return carry

        lax.fori_loop(0, n_chunks // 2 - 1, body, 0, unroll=False)

        load_idx(n_chunks - 1, idx1)
        fire_gather(idx1, rows1, sem1)
        wait_gather(idx0, rows0, sem0)
        writeout(n_chunks - 2, rows0)
        wait_gather(idx1, rows1, sem1)
        writeout(n_chunks - 1, rows1)

    return k


def kernel(input, tables):
    b, l, f = input.shape
    f2, v, d = tables.shape
    n = b * l * f
    idx_flat = input.reshape(n)
    tbl = tables.reshape(f2 * v, d)
    out = _gather_kernel(n, d, v, f, 1664)(tbl, idx_flat)
    return out.reshape(b, l, f, d)


# speedup vs baseline: 6.9837x; 2.0448x over previous
import functools

import jax
import jax.numpy as jnp
from jax import lax
from jax.experimental import pallas as pl
from jax.experimental.pallas import tpu as pltpu
from jax.experimental.pallas import tpu_sc as plsc

_NC = 2
_NS = 16
_NW = _NC * _NS

_LANES = 16


def _gather_kernel(f_fields, d_dim, v_rows, l_len, b_batch):
    n_pairs = f_fields * d_dim
    assert n_pairs % _NW == 0
    pairs_per_w = n_pairs // _NW

    mesh = plsc.VectorSubcoreMesh(core_axis_name="c", subcore_axis_name="s")

    @functools.partial(
        pl.kernel,
        mesh=mesh,
        compiler_params=pltpu.CompilerParams(needs_layout_passes=False),
        out_type=jax.ShapeDtypeStruct((l_len, f_fields, d_dim, b_batch), jnp.float32),
        scratch_types=[
            pltpu.VMEM((v_rows,), jnp.float32),
            pltpu.VMEM((b_batch,), jnp.int32),
            pltpu.VMEM((b_batch,), jnp.float32),
        ],
    )
    def k(tbl_hbm, in_hbm, out_hbm, row_v, idx_v, out_v):
        cid = lax.axis_index("c")
        sid = lax.axis_index("s")
        wid = sid * _NC + cid

        def pair_body(p, carry):
            f = p // d_dim
            d = p % d_dim
            pltpu.sync_copy(tbl_hbm.at[f, d], row_v)

            def l_body(l, carry2):
                pltpu.sync_copy(in_hbm.at[f, l], idx_v)

                def g_body(j, carry3):
                    for u in range(16):
                        sl = pl.ds((j * 16 + u) * _LANES, _LANES)
                        out_v[sl] = plsc.load_gather(row_v, [idx_v[sl]])
                    return carry3

                lax.fori_loop(0, b_batch // (16 * _LANES), g_body, 0,
                              unroll=False)
                pltpu.sync_copy(out_v, out_hbm.at[l, f, d])
                return carry2

            lax.fori_loop(0, l_len, l_body, 0, unroll=False)
            return carry

        lax.fori_loop(wid * pairs_per_w, (wid + 1) * pairs_per_w, pair_body, 0,
                      unroll=False)

    return k


def kernel(input, tables):
    b, l, f = input.shape
    f2, v, d = tables.shape
    tbl_t = jnp.transpose(tables, (0, 2, 1))
    in_t = jnp.transpose(input, (2, 1, 0))
    out_t = _gather_kernel(f, d, v, l, b)(tbl_t, in_t)
    return jnp.transpose(out_t, (3, 0, 1, 2))


# speedup vs baseline: 17.0624x; 2.4432x over previous
import functools

import jax
import jax.numpy as jnp
from jax import lax
from jax.experimental import pallas as pl
from jax.experimental.pallas import tpu as pltpu
from jax.experimental.pallas import tpu_sc as plsc

_NC = 2
_NS = 16
_NW = _NC * _NS

_LANES = 16


def _gather_kernel(f_fields, d_dim, v_rows, l_len, b_batch):
    n_pairs = f_fields * d_dim
    assert n_pairs % _NW == 0
    pairs_per_w = n_pairs // _NW

    mesh = plsc.VectorSubcoreMesh(core_axis_name="c", subcore_axis_name="s")

    @functools.partial(
        pl.kernel,
        mesh=mesh,
        compiler_params=pltpu.CompilerParams(needs_layout_passes=False),
        out_type=jax.ShapeDtypeStruct((l_len, f_fields, d_dim, b_batch), jnp.float32),
        scratch_types=[
            pltpu.VMEM((v_rows,), jnp.float32),
            pltpu.VMEM((b_batch,), jnp.int32),
            pltpu.VMEM((b_batch,), jnp.int32),
            pltpu.VMEM((b_batch,), jnp.float32),
            pltpu.VMEM((b_batch,), jnp.float32),
            pltpu.SemaphoreType.DMA,
            pltpu.SemaphoreType.DMA,
            pltpu.SemaphoreType.DMA,
            pltpu.SemaphoreType.DMA,
        ],
    )
    def k(tbl_hbm, in_hbm, out_hbm, row_v, idx0, idx1, out0, out1,
          sem_i0, sem_i1, sem_o0, sem_o1):
        cid = lax.axis_index("c")
        sid = lax.axis_index("s")
        wid = sid * _NC + cid
        idxs, outs = (idx0, idx1), (out0, out1)
        sem_i, sem_o = (sem_i0, sem_i1), (sem_o0, sem_o1)

        def pair_body(p, carry):
            f = p // d_dim
            d = p % d_dim
            pltpu.sync_copy(tbl_hbm.at[f, d], row_v)
            pltpu.sync_copy(in_hbm.at[f, 0], idxs[0])
            for l in range(l_len):
                a, b = l % 2, (l + 1) % 2
                if l + 1 < l_len:
                    pltpu.make_async_copy(
                        in_hbm.at[f, l + 1], idxs[b], sem_i[b]).start()
                if l >= 1:
                    pltpu.make_async_copy(
                        in_hbm.at[f, l], idxs[a], sem_i[a]).wait()
                if l >= 2:
                    pltpu.make_async_copy(
                        outs[a], out_hbm.at[l, f, d], sem_o[a]).wait()

                idx_v, out_v = idxs[a], outs[a]

                @plsc.parallel_loop(0, b_batch, step=_LANES, unroll=8)
                def g_body(i):
                    sl = pl.ds(i, _LANES)
                    out_v[sl] = plsc.load_gather(row_v, [idx_v[sl]])

                pltpu.make_async_copy(
                    outs[a], out_hbm.at[l, f, d], sem_o[a]).start()
            pltpu.make_async_copy(
                outs[0], out_hbm.at[l_len - 2, f, d], sem_o[0]).wait()
            pltpu.make_async_copy(
                outs[1], out_hbm.at[l_len - 1, f, d], sem_o[1]).wait()
            return carry

        lax.fori_loop(wid * pairs_per_w, (wid + 1) * pairs_per_w, pair_body, 0,
                      unroll=False)

    return k


def kernel(input, tables):
    b, l, f = input.shape
    f2, v, d = tables.shape
    tbl_t = jnp.transpose(tables, (0, 2, 1))
    in_t = jnp.transpose(input, (2, 1, 0))
    out_t = _gather_kernel(f, d, v, l, b)(tbl_t, in_t)
    return jnp.transpose(out_t, (3, 0, 1, 2))
